# baseline (speedup 1.0000x reference)
import jax
import jax.numpy as jnp
from jax import lax
from jax.experimental import pallas as pl
from jax.experimental.pallas import tpu as pltpu

D_MODEL = 512
FP_DIM = 2048
N_NODES = 16384
N_GRAPHS = 16
BLK = 4096
NBLK = N_NODES // BLK
NEG = -1e30


def _fused(seg_ref, x_ref, kfeat_ref, wq_ref, wk_ref, bv_ref,
           bih_ref, bhh_ref, wv_hbm, wih_hbm, whh_hbm,
           out_ref, kf_ref, proj_ref, m_ref, s_ref, acc_ref,
           wv_s, wih_s, whh_s, sem_v, sem_ih, sem_hh):
    i = pl.program_id(0)

    @pl.when(i == 0)
    def _prologue():
        pltpu.make_async_copy(wv_hbm, wv_s, sem_v).start()
        pltpu.make_async_copy(wih_hbm, wih_s, sem_ih).start()
        pltpu.make_async_copy(whh_hbm, whh_s, sem_hh).start()
        kf = lax.dot_general(kfeat_ref[...], wq_ref[...],
                             (((1,), (1,)), ((), ())),
                             preferred_element_type=jnp.float32)
        kf_ref[...] = kf
        proj_ref[...] = jnp.dot(kf, wk_ref[...],
                                preferred_element_type=jnp.float32) * (
                                    1.0 / (D_MODEL ** 0.5))
        m_ref[...] = jnp.full((N_GRAPHS, 1), NEG, jnp.float32)
        s_ref[...] = jnp.zeros((N_GRAPHS, 1), jnp.float32)
        acc_ref[...] = jnp.zeros((N_GRAPHS, D_MODEL), jnp.float32)

    x = x_ref[...]
    seg = seg_ref[0]
    xh = x.astype(jnp.bfloat16)
    xl = (x - xh.astype(jnp.float32)).astype(jnp.bfloat16)
    p = proj_ref[...]
    ph = p.astype(jnp.bfloat16)
    plo = (p - ph.astype(jnp.float32)).astype(jnp.bfloat16)

    def _dgt(a, b):
        return lax.dot_general(a, b, (((1,), (1,)), ((), ())),
                               preferred_element_type=jnp.float32)

    st = _dgt(ph, xh) + (_dgt(ph, xl) + _dgt(plo, xh))
    gids = lax.broadcasted_iota(jnp.int32, (N_GRAPHS, BLK), 0)
    mask = jnp.broadcast_to(seg, (N_GRAPHS, BLK)) == gids
    sm = jnp.where(mask, st, NEG)
    bm = jnp.max(sm, axis=1, keepdims=True)
    m_old = m_ref[...]
    m_new = jnp.maximum(m_old, bm)
    scale = jnp.exp(m_old - m_new)
    e = jnp.where(mask, jnp.exp(st - m_new), 0.0)
    s_ref[...] = s_ref[...] * scale + jnp.sum(e, axis=1, keepdims=True)
    m_ref[...] = m_new
    eh = e.astype(jnp.bfloat16)
    el = (e - eh.astype(jnp.float32)).astype(jnp.bfloat16)

    def _dg(a, b):
        return lax.dot_general(a, b, (((1,), (0,)), ((), ())),
                               preferred_element_type=jnp.float32)

    acc_ref[...] = acc_ref[...] * scale + (
        _dg(eh, xh) + (_dg(eh, xl) + _dg(el, xh)))

    @pl.when(i == NBLK - 1)
    def _epilogue():
        pltpu.make_async_copy(wv_hbm, wv_s, sem_v).wait()
        pltpu.make_async_copy(wih_hbm, wih_s, sem_ih).wait()
        pltpu.make_async_copy(whh_hbm, whh_s, sem_hh).wait()
        s = s_ref[...]
        has = (s > 0.0).astype(jnp.float32)
        pooled = acc_ref[...] / jnp.where(s > 0.0, s, 1.0)
        out = lax.dot_general(pooled, wv_s[...], (((1,), (1,)), ((), ())),
                              preferred_element_type=jnp.float32)
        out = out + has * bv_ref[...]
        kf = kf_ref[...]
        gi = lax.dot_general(out, wih_s[...], (((1,), (1,)), ((), ())),
                             preferred_element_type=jnp.float32) + bih_ref[...]
        gh = lax.dot_general(kf, whh_s[...], (((1,), (1,)), ((), ())),
                             preferred_element_type=jnp.float32) + bhh_ref[...]
        i_r = gi[:, :D_MODEL]
        i_z = gi[:, D_MODEL:2 * D_MODEL]
        i_n = gi[:, 2 * D_MODEL:]
        h_r = gh[:, :D_MODEL]
        h_z = gh[:, D_MODEL:2 * D_MODEL]
        h_n = gh[:, 2 * D_MODEL:]
        r = jax.nn.sigmoid(i_r + h_r)
        z = jax.nn.sigmoid(i_z + h_z)
        n = jnp.tanh(i_n + r * h_n)
        out_ref[...] = (1.0 - z) * n + z * kf


@jax.jit
def kernel(node_feature, k_feature, segment_ids, W_q, W_k, W_v, b_v,
           W_ih, W_hh, b_ih, b_hh):
    seg = segment_ids.astype(jnp.int32).reshape(NBLK, 1, BLK)
    bv2 = b_v.reshape(1, D_MODEL)
    bih2 = b_ih.reshape(1, 3 * D_MODEL)
    bhh2 = b_hh.reshape(1, 3 * D_MODEL)

    fixed = lambda i: (0, 0)
    out = pl.pallas_call(
        _fused,
        grid=(NBLK,),
        in_specs=[
            pl.BlockSpec((1, 1, BLK), lambda i: (i, 0, 0)),
            pl.BlockSpec((BLK, D_MODEL), lambda i: (i, 0)),
            pl.BlockSpec((N_GRAPHS, FP_DIM), fixed),
            pl.BlockSpec((D_MODEL, FP_DIM), fixed),
            pl.BlockSpec((D_MODEL, D_MODEL), fixed),
            pl.BlockSpec((1, D_MODEL), fixed),
            pl.BlockSpec((1, 3 * D_MODEL), fixed),
            pl.BlockSpec((1, 3 * D_MODEL), fixed),
            pl.BlockSpec(memory_space=pltpu.HBM),
            pl.BlockSpec(memory_space=pltpu.HBM),
            pl.BlockSpec(memory_space=pltpu.HBM),
        ],
        out_specs=pl.BlockSpec((N_GRAPHS, D_MODEL), fixed),
        out_shape=jax.ShapeDtypeStruct((N_GRAPHS, D_MODEL), jnp.float32),
        scratch_shapes=[
            pltpu.VMEM((N_GRAPHS, D_MODEL), jnp.float32),
            pltpu.VMEM((N_GRAPHS, D_MODEL), jnp.float32),
            pltpu.VMEM((N_GRAPHS, 1), jnp.float32),
            pltpu.VMEM((N_GRAPHS, 1), jnp.float32),
            pltpu.VMEM((N_GRAPHS, D_MODEL), jnp.float32),
            pltpu.VMEM((D_MODEL, D_MODEL), jnp.float32),
            pltpu.VMEM((3 * D_MODEL, D_MODEL), jnp.float32),
            pltpu.VMEM((3 * D_MODEL, D_MODEL), jnp.float32),
            pltpu.SemaphoreType.DMA,
            pltpu.SemaphoreType.DMA,
            pltpu.SemaphoreType.DMA,
        ],
        compiler_params=pltpu.CompilerParams(
            dimension_semantics=("arbitrary",)),
    )(seg, node_feature, k_feature, W_q, W_k, bv2, bih2, bhh2,
      W_v, W_ih, W_hh)
    return out


# speedup vs baseline: 1.5209x; 1.5209x over previous
import jax
import jax.numpy as jnp
from jax import lax
from jax.experimental import pallas as pl
from jax.experimental.pallas import tpu as pltpu

D_MODEL = 512
FP_DIM = 2048
N_NODES = 16384
N_GRAPHS = 16
BLK = 8192
NBLK = N_NODES // BLK
NEG = -1e30


def _fused(seg_ref, x_ref, kfeat_ref, wq_ref, wk_ref, bv_ref,
           bih_ref, bhh_ref, wv_hbm, wih_hbm, whh_hbm,
           out_ref, kf_ref, proj_ref, m_ref, s_ref, acc_ref,
           wv_s, wih_s, whh_s, sem_v, sem_ih, sem_hh):
    i = pl.program_id(0)

    @pl.when(i == 0)
    def _prologue():
        pltpu.make_async_copy(wv_hbm, wv_s, sem_v).start()
        pltpu.make_async_copy(wih_hbm, wih_s, sem_ih).start()
        pltpu.make_async_copy(whh_hbm, whh_s, sem_hh).start()
        kf = lax.dot_general(kfeat_ref[...], wq_ref[...],
                             (((1,), (1,)), ((), ())),
                             preferred_element_type=jnp.float32)
        kf_ref[...] = kf
        proj_ref[...] = jnp.dot(kf, wk_ref[...],
                                preferred_element_type=jnp.float32) * (
                                    1.0 / (D_MODEL ** 0.5))
        m_ref[...] = jnp.full((N_GRAPHS, 1), NEG, jnp.float32)
        s_ref[...] = jnp.zeros((N_GRAPHS, 1), jnp.float32)
        acc_ref[...] = jnp.zeros((N_GRAPHS, D_MODEL), jnp.float32)

    x = x_ref[...]
    seg = seg_ref[0]
    st = lax.dot_general(proj_ref[...], x, (((1,), (1,)), ((), ())),
                         preferred_element_type=jnp.float32)
    gids = lax.broadcasted_iota(jnp.int32, (N_GRAPHS, BLK), 0)
    mask = jnp.broadcast_to(seg, (N_GRAPHS, BLK)) == gids
    sm = jnp.where(mask, st, NEG)
    bm = jnp.max(sm, axis=1, keepdims=True)
    m_old = m_ref[...]
    m_new = jnp.maximum(m_old, bm)
    scale = jnp.exp(m_old - m_new)
    e = jnp.where(mask, jnp.exp(st - m_new), 0.0)
    s_ref[...] = s_ref[...] * scale + jnp.sum(e, axis=1, keepdims=True)
    m_ref[...] = m_new
    acc_ref[...] = acc_ref[...] * scale + jnp.dot(
        e, x, preferred_element_type=jnp.float32)

    @pl.when(i == NBLK - 1)
    def _epilogue():
        pltpu.make_async_copy(wv_hbm, wv_s, sem_v).wait()
        pltpu.make_async_copy(wih_hbm, wih_s, sem_ih).wait()
        pltpu.make_async_copy(whh_hbm, whh_s, sem_hh).wait()
        s = s_ref[...]
        has = (s > 0.0).astype(jnp.float32)
        pooled = acc_ref[...] / jnp.where(s > 0.0, s, 1.0)
        out = lax.dot_general(pooled, wv_s[...], (((1,), (1,)), ((), ())),
                              preferred_element_type=jnp.float32)
        out = out + has * bv_ref[...]
        kf = kf_ref[...]
        gi = lax.dot_general(out, wih_s[...], (((1,), (1,)), ((), ())),
                             preferred_element_type=jnp.float32) + bih_ref[...]
        gh = lax.dot_general(kf, whh_s[...], (((1,), (1,)), ((), ())),
                             preferred_element_type=jnp.float32) + bhh_ref[...]
        i_r = gi[:, :D_MODEL]
        i_z = gi[:, D_MODEL:2 * D_MODEL]
        i_n = gi[:, 2 * D_MODEL:]
        h_r = gh[:, :D_MODEL]
        h_z = gh[:, D_MODEL:2 * D_MODEL]
        h_n = gh[:, 2 * D_MODEL:]
        r = jax.nn.sigmoid(i_r + h_r)
        z = jax.nn.sigmoid(i_z + h_z)
        n = jnp.tanh(i_n + r * h_n)
        out_ref[...] = (1.0 - z) * n + z * kf


@jax.jit
def kernel(node_feature, k_feature, segment_ids, W_q, W_k, W_v, b_v,
           W_ih, W_hh, b_ih, b_hh):
    seg = segment_ids.astype(jnp.int32).reshape(NBLK, 1, BLK)
    bv2 = b_v.reshape(1, D_MODEL)
    bih2 = b_ih.reshape(1, 3 * D_MODEL)
    bhh2 = b_hh.reshape(1, 3 * D_MODEL)

    fixed = lambda i: (0, 0)
    out = pl.pallas_call(
        _fused,
        grid=(NBLK,),
        in_specs=[
            pl.BlockSpec((1, 1, BLK), lambda i: (i, 0, 0)),
            pl.BlockSpec((BLK, D_MODEL), lambda i: (i, 0)),
            pl.BlockSpec((N_GRAPHS, FP_DIM), fixed),
            pl.BlockSpec((D_MODEL, FP_DIM), fixed),
            pl.BlockSpec((D_MODEL, D_MODEL), fixed),
            pl.BlockSpec((1, D_MODEL), fixed),
            pl.BlockSpec((1, 3 * D_MODEL), fixed),
            pl.BlockSpec((1, 3 * D_MODEL), fixed),
            pl.BlockSpec(memory_space=pltpu.HBM),
            pl.BlockSpec(memory_space=pltpu.HBM),
            pl.BlockSpec(memory_space=pltpu.HBM),
        ],
        out_specs=pl.BlockSpec((N_GRAPHS, D_MODEL), fixed),
        out_shape=jax.ShapeDtypeStruct((N_GRAPHS, D_MODEL), jnp.float32),
        scratch_shapes=[
            pltpu.VMEM((N_GRAPHS, D_MODEL), jnp.float32),
            pltpu.VMEM((N_GRAPHS, D_MODEL), jnp.float32),
            pltpu.VMEM((N_GRAPHS, 1), jnp.float32),
            pltpu.VMEM((N_GRAPHS, 1), jnp.float32),
            pltpu.VMEM((N_GRAPHS, D_MODEL), jnp.float32),
            pltpu.VMEM((D_MODEL, D_MODEL), jnp.float32),
            pltpu.VMEM((3 * D_MODEL, D_MODEL), jnp.float32),
            pltpu.VMEM((3 * D_MODEL, D_MODEL), jnp.float32),
            pltpu.SemaphoreType.DMA,
            pltpu.SemaphoreType.DMA,
            pltpu.SemaphoreType.DMA,
        ],
        compiler_params=pltpu.CompilerParams(
            dimension_semantics=("arbitrary",)),
    )(seg, node_feature, k_feature, W_q, W_k, bv2, bih2, bhh2,
      W_v, W_ih, W_hh)
    return out
